# baseline (speedup 1.0000x reference)
import functools

import jax
import jax.numpy as jnp
from jax import lax
from jax.experimental import pallas as pl
from jax.experimental.pallas import tpu as pltpu
from jax.experimental.pallas import tpu_sc as plsc

BATCH = 16384
EMB = 64
NUM_CORES = 2
NUM_SUBCORES = 16
NUM_WORKERS = NUM_CORES * NUM_SUBCORES
BPW = BATCH // NUM_WORKERS
GROUPS = BPW // 16


def _recommender_sc(user_emb, movie_emb, user_bias, movie_bias, user_idx, movie_idx):
    mesh = plsc.VectorSubcoreMesh(core_axis_name="c", subcore_axis_name="s")

    @functools.partial(
        pl.kernel,
        mesh=mesh,
        compiler_params=pltpu.CompilerParams(
            needs_layout_passes=False, use_tc_tiling_on_sc=False),
        out_type=jax.ShapeDtypeStruct((BATCH,), jnp.float32),
        scratch_types=[
            pltpu.VMEM((BPW,), jnp.int32),
            pltpu.VMEM((BPW,), jnp.int32),
            pltpu.VMEM((BPW, EMB), jnp.float32),
            pltpu.VMEM((BPW, EMB), jnp.float32),
            pltpu.VMEM((BPW,), jnp.float32),
            pltpu.VMEM((BPW,), jnp.float32),
            pltpu.VMEM((BPW,), jnp.float32),
            pltpu.VMEM((16, 16), jnp.float32),
            pltpu.SemaphoreType.DMA,
        ],
    )
    def k(uemb_hbm, memb_hbm, ubias_hbm, mbias_hbm, uidx_hbm, midx_hbm, out_hbm,
          uidx_v, midx_v, urows_v, mrows_v, ub_v, mb_v, out_v, tr_v, sem):
        wid = lax.axis_index("s") * NUM_CORES + lax.axis_index("c")
        base = wid * BPW

        pltpu.sync_copy(uidx_hbm.at[pl.ds(base, BPW)], uidx_v)
        pltpu.sync_copy(midx_hbm.at[pl.ds(base, BPW)], midx_v)

        c1 = pltpu.async_copy(uemb_hbm.at[uidx_v], urows_v, sem)
        c2 = pltpu.async_copy(memb_hbm.at[midx_v], mrows_v, sem)
        c3 = pltpu.async_copy(ubias_hbm.at[uidx_v], ub_v, sem)
        c4 = pltpu.async_copy(mbias_hbm.at[midx_v], mb_v, sem)
        c1.wait()
        c2.wait()
        c3.wait()
        c4.wait()

        lane = lax.iota(jnp.int32, 16)
        col15 = lane * 0 + 15

        @pl.loop(0, GROUPS)
        def _(g):
            b0 = g * 16
            for i in range(16):
                b = b0 + i
                acc = urows_v[b, pl.ds(0, 16)] * mrows_v[b, pl.ds(0, 16)]
                for c in range(1, 4):
                    acc = acc + urows_v[b, pl.ds(c * 16, 16)] * mrows_v[b, pl.ds(c * 16, 16)]
                tr_v[i, :] = jnp.cumsum(acc)
            hsum = plsc.load_gather(tr_v, [lane, col15])
            res = hsum + ub_v[pl.ds(b0, 16)] + mb_v[pl.ds(b0, 16)] + 3.5
            out_v[pl.ds(b0, 16)] = res

        pltpu.sync_copy(out_v, out_hbm.at[pl.ds(base, BPW)])

    return k(user_emb, movie_emb, user_bias, movie_bias, user_idx, movie_idx)


def kernel(user_idx, movie_idx, user_embedding, movie_embedding, user_bias, movie_bias):
    return _recommender_sc(
        user_embedding,
        movie_embedding,
        user_bias.reshape(-1),
        movie_bias.reshape(-1),
        user_idx.astype(jnp.int32),
        movie_idx.astype(jnp.int32),
    )
